# Initial kernel scaffold; baseline (speedup 1.0000x reference)
import jax
import jax.numpy as jnp
from jax.experimental import pallas as pl


def kernel(x, edge_index, batch, W1, b1, W2, b2):
    raise NotImplementedError("write your pallas kernel here")


# speedup vs baseline: 32.9084x; 32.9084x over previous
import functools

import jax
import jax.numpy as jnp
from jax import lax
from jax.experimental import pallas as pl
from jax.experimental.pallas import tpu as pltpu
from jax.experimental.pallas import tpu_sc as plsc

NN = 10000
NP = 10240
EE = 320000
NC, NS = 2, 16
NT = NC * NS
CH = 128
NCHUNK = 79
EPT = CH * NCHUNK
EPAD = NT * EPT
FW = 16
RPT = NP // NS
NG = 64
BN = 2048
GRID = NP // BN

_mesh = plsc.VectorSubcoreMesh(
    core_axis_name="c", subcore_axis_name="s", num_cores=NC, num_subcores=NS
)

_sc_params = pltpu.CompilerParams(use_tc_tiling_on_sc=False)


@functools.partial(
    pl.kernel,
    out_type=jax.ShapeDtypeStruct((NC, NP, FW), jnp.float32),
    mesh=_mesh,
    scratch_types=[
        pltpu.VMEM((NCHUNK, CH), jnp.int32),
        pltpu.VMEM((CH, FW), jnp.float32),
        pltpu.VMEM((RPT, FW), jnp.float32),
        pltpu.VMEM_SHARED((NP, FW), jnp.float32),
    ],
    compiler_params=_sc_params,
)
def _sc_degree(dst3, zeros_hbm, ones_hbm, out, idx_v, ones_v, buf_v, acc_sh):
    cid = lax.axis_index("c")
    sid = lax.axis_index("s")
    wid = cid * NS + sid
    pltpu.sync_copy(dst3.at[wid], idx_v)
    pltpu.sync_copy(ones_hbm, ones_v)
    pltpu.sync_copy(zeros_hbm.at[pl.ds(sid * RPT, RPT)], buf_v)
    pltpu.sync_copy(buf_v, acc_sh.at[pl.ds(sid * RPT, RPT)])
    plsc.subcore_barrier()

    def body(j, carry):
        pltpu.sync_copy(ones_v, acc_sh.at[idx_v.at[j]], add=True)
        return carry

    lax.fori_loop(0, NCHUNK, body, 0)
    plsc.subcore_barrier()
    pltpu.sync_copy(acc_sh.at[pl.ds(sid * RPT, RPT)], buf_v)
    pltpu.sync_copy(buf_v, out.at[cid, pl.ds(sid * RPT, RPT)])


@functools.partial(
    pl.kernel,
    out_type=jax.ShapeDtypeStruct((NC, NP, FW), jnp.float32),
    mesh=_mesh,
    scratch_types=[
        pltpu.VMEM((NCHUNK, CH), jnp.int32),
        pltpu.VMEM((NCHUNK, CH), jnp.int32),
        pltpu.VMEM((CH, FW), jnp.float32),
        pltpu.VMEM((RPT, FW), jnp.float32),
        pltpu.VMEM_SHARED((NP, FW), jnp.float32),
        pltpu.SemaphoreType.DMA,
    ],
    compiler_params=_sc_params,
)
def _sc_aggregate(src3, dst3, table, zeros_hbm, out,
                  si_v, di_v, rows_v, buf_v, acc_sh, sem):
    cid = lax.axis_index("c")
    sid = lax.axis_index("s")
    wid = cid * NS + sid
    pltpu.sync_copy(src3.at[wid], si_v)
    pltpu.sync_copy(dst3.at[wid], di_v)
    pltpu.sync_copy(zeros_hbm.at[pl.ds(sid * RPT, RPT)], buf_v)
    pltpu.sync_copy(buf_v, acc_sh.at[pl.ds(sid * RPT, RPT)])
    plsc.subcore_barrier()

    def body(j, carry):
        pltpu.async_copy(table.at[si_v.at[j]], rows_v, sem).wait()
        pltpu.sync_copy(rows_v, acc_sh.at[di_v.at[j]], add=True)
        return carry

    lax.fori_loop(0, NCHUNK, body, 0)
    plsc.subcore_barrier()
    pltpu.sync_copy(acc_sh.at[pl.ds(sid * RPT, RPT)], buf_v)
    pltpu.sync_copy(buf_v, out.at[cid, pl.ds(sid * RPT, RPT)])


def _tc1_body(x_ref, w1_ref, degp_ref, g1_ref, dinv_ref):
    d = degp_ref[...]
    deg = d[0] + d[1] + 1.0
    dinv = lax.rsqrt(deg)
    h = jnp.dot(x_ref[...], w1_ref[...], preferred_element_type=jnp.float32)
    g1_ref[...] = dinv * h
    dinv_ref[...] = dinv


def _tc2_body(p_ref, g1_ref, dinv_ref, b1_ref, w2_ref, g2_ref):
    p = p_ref[...]
    dinv = dinv_ref[...]
    t = dinv * (p[0] + p[1] + g1_ref[...]) + b1_ref[...]
    h1p = jnp.maximum(t, 0.0)
    g2_ref[...] = dinv * jnp.dot(
        h1p, w2_ref[...], preferred_element_type=jnp.float32
    )


def _tc3_body(p_ref, g2_ref, dinv_ref, b2_ref, batch_ref, out_ref, acc_ref):
    i = pl.program_id(0)

    @pl.when(i == 0)
    def _init():
        acc_ref[...] = jnp.zeros_like(acc_ref)

    p = p_ref[...]
    nodes = dinv_ref[...] * (p[0] + p[1] + g2_ref[...]) + b2_ref[...]
    col = lax.broadcasted_iota(jnp.int32, (BN, FW), 1)
    nodes = jnp.where(col < 10, nodes, jnp.where(col == 10, 1.0, 0.0))
    bvals = batch_ref[...].reshape(1, BN)
    gid = lax.broadcasted_iota(jnp.int32, (NG, BN), 0)
    mask = (gid == jnp.broadcast_to(bvals, (NG, BN))).astype(jnp.float32)
    acc_ref[...] += jnp.dot(mask, nodes, preferred_element_type=jnp.float32)

    @pl.when(i == GRID - 1)
    def _finish():
        a = acc_ref[...]
        cnt = jnp.maximum(a[:, 10:11], 1.0)
        v = a / cnt
        colv = lax.broadcasted_iota(jnp.int32, (NG, FW), 1)
        m = jnp.max(jnp.where(colv < 10, v, -1e30), axis=1, keepdims=True)
        e = jnp.where(colv < 10, jnp.exp(v - m), 0.0)
        lse = jnp.log(jnp.sum(e, axis=1, keepdims=True))
        out_ref[...] = (v - m - lse)[:, :10]


_tc1 = pl.pallas_call(
    _tc1_body,
    grid=(GRID,),
    in_specs=[
        pl.BlockSpec((BN, 128), lambda i: (i, 0)),
        pl.BlockSpec((128, FW), lambda i: (0, 0)),
        pl.BlockSpec((NC, BN, FW), lambda i: (0, i, 0)),
    ],
    out_specs=[
        pl.BlockSpec((BN, FW), lambda i: (i, 0)),
        pl.BlockSpec((BN, FW), lambda i: (i, 0)),
    ],
    out_shape=[
        jax.ShapeDtypeStruct((NP, FW), jnp.float32),
        jax.ShapeDtypeStruct((NP, FW), jnp.float32),
    ],
)

_tc2 = pl.pallas_call(
    _tc2_body,
    grid=(GRID,),
    in_specs=[
        pl.BlockSpec((NC, BN, FW), lambda i: (0, i, 0)),
        pl.BlockSpec((BN, FW), lambda i: (i, 0)),
        pl.BlockSpec((BN, FW), lambda i: (i, 0)),
        pl.BlockSpec((1, FW), lambda i: (0, 0)),
        pl.BlockSpec((FW, FW), lambda i: (0, 0)),
    ],
    out_specs=pl.BlockSpec((BN, FW), lambda i: (i, 0)),
    out_shape=jax.ShapeDtypeStruct((NP, FW), jnp.float32),
)

_tc3 = pl.pallas_call(
    _tc3_body,
    grid=(GRID,),
    in_specs=[
        pl.BlockSpec((NC, BN, FW), lambda i: (0, i, 0)),
        pl.BlockSpec((BN, FW), lambda i: (i, 0)),
        pl.BlockSpec((BN, FW), lambda i: (i, 0)),
        pl.BlockSpec((1, FW), lambda i: (0, 0)),
        pl.BlockSpec((1, 1, BN), lambda i: (i, 0, 0)),
    ],
    out_specs=pl.BlockSpec((NG, 10), lambda i: (0, 0)),
    out_shape=jax.ShapeDtypeStruct((NG, 10), jnp.float32),
    scratch_shapes=[pltpu.VMEM((NG, FW), jnp.float32)],
)


@jax.jit
def kernel(x, edge_index, batch, W1, b1, W2, b2):
    x = x.astype(jnp.float32)
    ei = edge_index.astype(jnp.int32)
    batch = batch.astype(jnp.int32)

    pad_e = jnp.full((EPAD - EE,), NN, jnp.int32)
    src3 = jnp.concatenate([ei[0], pad_e]).reshape(NT, NCHUNK, CH)
    dst3 = jnp.concatenate([ei[1], pad_e]).reshape(NT, NCHUNK, CH)
    xp = jnp.concatenate([x, jnp.zeros((NP - NN, x.shape[1]), jnp.float32)])
    zeros_tab = jnp.zeros((NP, FW), jnp.float32)
    ones_rows = jnp.ones((CH, FW), jnp.float32)
    batch3 = jnp.concatenate(
        [batch, jnp.full((NP - NN,), NG, jnp.int32)]
    ).reshape(GRID, 1, BN)
    b1r = b1.astype(jnp.float32).reshape(1, FW)
    w2p = jnp.pad(W2.astype(jnp.float32), ((0, 0), (0, FW - W2.shape[1])))
    b2r = jnp.pad(b2.astype(jnp.float32), (0, FW - b2.shape[0])).reshape(1, FW)

    degp = _sc_degree(dst3, zeros_tab, ones_rows)
    g1, dinv16 = _tc1(xp, W1.astype(jnp.float32), degp)
    p1 = _sc_aggregate(src3, dst3, g1, zeros_tab)
    g2 = _tc2(p1, g1, dinv16, b1r, w2p)
    p2 = _sc_aggregate(src3, dst3, g2, zeros_tab)
    return _tc3(p2, g2, dinv16, b2r, batch3)
